# baseline (speedup 1.0000x reference)
import functools

import jax
import jax.numpy as jnp
from jax.experimental import pallas as pl
from jax.experimental.pallas import tpu as pltpu
from jax.experimental.pallas import tpu_sc as plsc

B, T, D = 4, 128, 512
BS = 16
NBLK = T // BS
_F32 = jnp.float32
_NEG_INF = float("-inf")
_CONTRACT_LAST = (((1,), (1,)), ((), ()))
_CONTRACT_NATIVE = (((1,), (0,)), ((), ()))


def _dot(a, b, dims):
    return jax.lax.dot_general(a, b, dims, preferred_element_type=_F32)


def _front_body(x_ref, wit_ref, bit_ref, wq_ref, bq_ref, wa_ref,
                ft_ref, qw_ref):
    x = x_ref[:]
    feat = jnp.maximum(_dot(x, wit_ref[:], _CONTRACT_LAST) + bit_ref[:], 0.0)
    ft_ref[:] = feat
    q = _dot(feat, wq_ref[:], _CONTRACT_LAST) + bq_ref[:]
    qw_ref[:] = q * wa_ref[:]


_SC_CACHE = {}


def _lstar_sc(miT3):
    key = miT3.shape
    if key not in _SC_CACHE:
        info = plsc.get_sparse_core_info()
        nw = info.num_cores * info.num_subcores
        lanes = info.num_lanes
        rpw = (T * B) // nw
        assert rpw == lanes and miT3.shape == (nw, rpw * T)
        mesh = plsc.VectorSubcoreMesh(core_axis_name="c", subcore_axis_name="s")

        @functools.partial(
            pl.kernel, mesh=mesh,
            out_type=jax.ShapeDtypeStruct((T * B,), _F32),
            scratch_types=[
                pltpu.VMEM((rpw * T,), jnp.int32),
                pltpu.VMEM((rpw,), jnp.int32),
                pltpu.VMEM((rpw,), _F32),
            ],
        )
        def lstar_kernel(mi_hbm, tv_hbm, out_hbm, buf_v, tv_v, res_v):
            wid = jax.lax.axis_index("s") * info.num_cores \
                + jax.lax.axis_index("c")
            base = wid * rpw
            pltpu.sync_copy(mi_hbm.at[wid], buf_v)
            pltpu.sync_copy(tv_hbm.at[pl.ds(base, rpw)], tv_v)
            tvec = tv_v[:]
            one = jax.lax.broadcasted_iota(jnp.int32, (lanes,), 0) * 0 + 1
            neg1 = one * 0 - 1

            def body(p, carry):
                acc, pv = carry
                v = buf_v[pl.ds(p * lanes, lanes)]
                cond = (v == one) & (tvec > pv)
                acc = jnp.maximum(acc, jnp.where(cond, pv, neg1))
                return (acc, pv + one)

            acc, _ = jax.lax.fori_loop(0, T, body, (neg1, one * 0))
            res_v[:] = acc.astype(_F32)
            pltpu.sync_copy(res_v, out_hbm.at[pl.ds(base, rpw)])

        _SC_CACHE[key] = lstar_kernel

    tvall = (jnp.arange(T * B, dtype=jnp.int32) // B)
    return _SC_CACHE[key](miT3, tvall)


def _sse_body(ft_in_ref, qw_in_ref, ls_ref, mi0_ref, um_ref, wo_ref, bo_ref,
              out_ref, v_ref, upre_ref, stat_ref):
    lane = jax.lax.broadcasted_iota(jnp.int32, (1, T), 1)
    sub = jax.lax.broadcasted_iota(jnp.int32, (T * B, 1), 0)
    tid = sub // B
    bcol = sub % B
    feat = ft_in_ref[:]

    mi4 = mi0_ref[0:B, :]
    c = (mi4 != mi4[:, 0:1]) & (lane >= 1)
    kidx4 = jnp.min(jnp.where(c, lane, 2 * T), axis=1, keepdims=True)
    sel = tid < 0
    for b in range(B):
        sel = sel | ((bcol == b) & ((tid == 0) | (tid == kidx4[b:b + 1])))
    v_ref[:] = jnp.where(sel, feat, 0.0)

    alive = jnp.ones((B, 1), _F32)
    lane_blk = jax.lax.broadcasted_iota(jnp.int32, (1, BS * B), 1)
    brow4 = jax.lax.broadcasted_iota(jnp.int32, (B, 1), 0)

    def cat4(ref, base):
        return jnp.concatenate(
            [ref[pl.ds(base + i, 1), :] for i in range(B)], axis=0)

    vprev = cat4(v_ref, 0)

    for k in range(NBLK):
        j0 = k * BS
        r0 = j0 * B
        lo = max(j0, 1)

        lstar = ls_ref[r0:r0 + BS * B, :]
        if j0 == 0:
            stat_ref[:] = jnp.concatenate(
                [lstar, jnp.full((BS * B, 1), _NEG_INF),
                 jnp.zeros((BS * B, 1), _F32)], axis=1)
            upre_ref[:] = jnp.zeros((BS * B, D), _F32)
        else:
            vpre = v_ref[0:r0, :]
            qw_blk = qw_in_ref[r0:r0 + BS * B, :]
            s_pre = _dot(qw_blk, vpre, _CONTRACT_LAST)
            lane_pre = jax.lax.broadcasted_iota(jnp.int32, (1, r0), 1)
            wpre = ((lane_pre % B == bcol[r0:r0 + BS * B, :])
                    & ((lane_pre // B).astype(_F32) >= lstar))
            m_pre = jnp.max(jnp.where(wpre, s_pre, _NEG_INF), axis=1,
                            keepdims=True)
            e_pre = jnp.where(wpre, jnp.exp(s_pre - m_pre), 0.0)
            stat_ref[:] = jnp.concatenate(
                [lstar, m_pre,
                 jnp.sum(e_pre, axis=1, keepdims=True)], axis=1)
            upre_ref[:] = _dot(e_pre, vpre, _CONTRACT_NATIVE)

        tc = lane_blk // B + j0
        tc_f = tc.astype(_F32)
        bc = lane_blk % B

        def step(j, carry):
            alive, vprev, m_ip, d_ip, e_ip, stats_cur, qw_cur = carry
            rp = (j - j0) * B
            v_blk = v_ref[r0:r0 + BS * B, :]
            u_ip = _dot(e_ip, v_blk, _CONTRACT_NATIVE)
            jn = jnp.minimum(j + 1, T - 1)
            qw_next = cat4(qw_in_ref, B * jn)
            s_next = _dot(qw_next, v_blk, _CONTRACT_LAST)

            u_pre = cat4(upre_ref, rp)
            fr = cat4(ft_in_ref, B * j)
            umv = cat4(um_ref, B * j)
            rpn = jnp.minimum(rp + B, (BS - 1) * B)
            stats_next = cat4(stat_ref, rpn)
            lstar4 = stats_cur[:, 0:1]
            m_pre = stats_cur[:, 1:2]
            d_pre = stats_cur[:, 2:3]

            corr_s = jnp.sum(qw_cur * vprev, axis=1, keepdims=True)
            corr_m = jnp.where(j > j0, corr_s, _NEG_INF)
            m_all = jnp.maximum(jnp.maximum(m_ip, corr_m), m_pre)
            c_pre = jnp.exp(m_pre - m_all)
            c_ip = jnp.exp(m_ip - m_all)
            e_c = jnp.exp(corr_m - m_all)
            num = c_pre * u_pre + c_ip * u_ip + e_c * vprev
            den = c_pre * d_pre + c_ip * d_ip + e_c
            v_att = jnp.tanh(num / den)
            alive = alive * umv
            vj = jnp.where(lstar4 >= 0.0, v_att, fr)
            vj = jnp.where(alive > 0, vj, jnp.where(kidx4 == j, fr, 0.0))
            for i in range(B):
                v_ref[pl.ds(B * j + i, 1), :] = vj[i:i + 1, :]

            lstar_n = stats_next[:, 0:1]
            wip = (bc == brow4) & (tc_f >= lstar_n) & (tc < j)
            m_ip_n = jnp.max(jnp.where(wip, s_next, _NEG_INF), axis=1,
                             keepdims=True)
            e_ip_n = jnp.where(wip, jnp.exp(s_next - m_ip_n), 0.0)
            d_ip_n = jnp.sum(e_ip_n, axis=1, keepdims=True)
            return (alive, vj, m_ip_n, d_ip_n, e_ip_n, stats_next, qw_next)

        carry = (alive, vprev,
                 jnp.full((B, 1), _NEG_INF, _F32),
                 jnp.zeros((B, 1), _F32),
                 jnp.zeros((B, BS * B), _F32),
                 cat4(stat_ref, (lo - j0) * B),
                 cat4(qw_in_ref, B * lo))
        carry = jax.lax.fori_loop(lo, j0 + BS, step, carry)
        alive, vprev = carry[0], carry[1]

    o = _dot(ft_in_ref[:], wo_ref[:], _CONTRACT_LAST) + bo_ref[:]
    out_ref[:] = jnp.maximum(o * v_ref[:], 0.0) + ft_in_ref[:]


def kernel(feature, mask_intra, umask, W_init_trans, b_init_trans,
           W_qinter, b_qinter, W_attn, b_attn, W_out, b_out):
    del b_attn
    x2 = feature.transpose(1, 0, 2).reshape(T * B, D)
    mi2 = mask_intra.astype(jnp.int32).transpose(1, 0, 2).reshape(T * B, T)
    nw = (T * B) // 16
    miT3 = mi2.reshape(nw, 16, T).transpose(0, 2, 1).reshape(nw, T * 16)
    mi0 = mask_intra.astype(jnp.int32)[:, 0, :]
    umr = umask.astype(_F32).T.reshape(T * B, 1)
    bit = b_init_trans.reshape(1, D)
    bq = b_qinter.reshape(1, D)
    bo = b_out.reshape(1, D)

    ft2, qw2 = pl.pallas_call(
        _front_body,
        out_shape=[jax.ShapeDtypeStruct((T * B, D), _F32)] * 2,
    )(x2, W_init_trans, bit, W_qinter, bq, W_attn)

    ls2 = _lstar_sc(miT3).reshape(T * B, 1)

    out2 = pl.pallas_call(
        _sse_body,
        out_shape=jax.ShapeDtypeStruct((T * B, D), _F32),
        scratch_shapes=[
            pltpu.VMEM((T * B, D), _F32),
            pltpu.VMEM((BS * B, D), _F32),
            pltpu.VMEM((BS * B, 3), _F32),
        ],
    )(ft2, qw2, ls2, mi0, umr, W_out, bo)
    return out2.reshape(T, B, D).transpose(1, 0, 2)


# speedup vs baseline: 1.0360x; 1.0360x over previous
import functools

import jax
import jax.numpy as jnp
from jax.experimental import pallas as pl
from jax.experimental.pallas import tpu as pltpu
from jax.experimental.pallas import tpu_sc as plsc

B, T, D = 4, 128, 512
BS = 16
NBLK = T // BS
_F32 = jnp.float32
_NEG_INF = float("-inf")
_CONTRACT_LAST = (((1,), (1,)), ((), ()))
_CONTRACT_NATIVE = (((1,), (0,)), ((), ()))


def _dot(a, b, dims):
    return jax.lax.dot_general(a, b, dims, preferred_element_type=_F32)


def _front_body(x_ref, wit_ref, bit_ref, wq_ref, bq_ref, wa_ref,
                ft_ref, qw_ref):
    x = x_ref[:]
    feat = jnp.maximum(_dot(x, wit_ref[:], _CONTRACT_LAST) + bit_ref[:], 0.0)
    ft_ref[:] = feat
    q = _dot(feat, wq_ref[:], _CONTRACT_LAST) + bq_ref[:]
    qw_ref[:] = q * wa_ref[:]


_SC_CACHE = {}


def _lstar_sc(miT3):
    key = miT3.shape
    if key not in _SC_CACHE:
        info = plsc.get_sparse_core_info()
        nw = info.num_cores * info.num_subcores
        lanes = info.num_lanes
        rpw = (T * B) // nw
        assert rpw == lanes and miT3.shape == (nw, rpw * T)
        mesh = plsc.VectorSubcoreMesh(core_axis_name="c", subcore_axis_name="s")

        @functools.partial(
            pl.kernel, mesh=mesh,
            out_type=jax.ShapeDtypeStruct((T * B,), _F32),
            scratch_types=[
                pltpu.VMEM((rpw * T,), jnp.int32),
                pltpu.VMEM((rpw,), jnp.int32),
                pltpu.VMEM((rpw,), _F32),
            ],
        )
        def lstar_kernel(mi_hbm, tv_hbm, out_hbm, buf_v, tv_v, res_v):
            wid = jax.lax.axis_index("s") * info.num_cores \
                + jax.lax.axis_index("c")
            base = wid * rpw
            pltpu.sync_copy(mi_hbm.at[wid], buf_v)
            pltpu.sync_copy(tv_hbm.at[pl.ds(base, rpw)], tv_v)
            tvec = tv_v[:]
            one = jax.lax.broadcasted_iota(jnp.int32, (lanes,), 0) * 0 + 1
            neg1 = one * 0 - 1

            def body(p, carry):
                acc, pv = carry
                v = buf_v[pl.ds(p * lanes, lanes)]
                cond = (v == one) & (tvec > pv)
                acc = jnp.maximum(acc, jnp.where(cond, pv, neg1))
                return (acc, pv + one)

            acc, _ = jax.lax.fori_loop(0, T, body, (neg1, one * 0))
            res_v[:] = acc.astype(_F32)
            pltpu.sync_copy(res_v, out_hbm.at[pl.ds(base, rpw)])

        _SC_CACHE[key] = lstar_kernel

    tvall = (jnp.arange(T * B, dtype=jnp.int32) // B)
    return _SC_CACHE[key](miT3, tvall)


def _sse_body(x_ref, ls_ref, mi0_ref, um_ref, wit_ref, bit_ref, wq_ref,
              bq_ref, wa_ref, wo_ref, bo_ref, out_ref,
              v_ref, ft_in_ref, qw_in_ref, upre_ref, stat_ref):
    x = x_ref[:]
    feat = jnp.maximum(_dot(x, wit_ref[:], _CONTRACT_LAST) + bit_ref[:], 0.0)
    ft_in_ref[:] = feat
    q = _dot(feat, wq_ref[:], _CONTRACT_LAST) + bq_ref[:]
    qw_in_ref[:] = q * wa_ref[:]

    lane = jax.lax.broadcasted_iota(jnp.int32, (1, T), 1)
    sub = jax.lax.broadcasted_iota(jnp.int32, (T * B, 1), 0)
    tid = sub // B
    bcol = sub % B

    mi4 = mi0_ref[0:B, :]
    c = (mi4 != mi4[:, 0:1]) & (lane >= 1)
    kidx4 = jnp.min(jnp.where(c, lane, 2 * T), axis=1, keepdims=True)
    sel = tid < 0
    for b in range(B):
        sel = sel | ((bcol == b) & ((tid == 0) | (tid == kidx4[b:b + 1])))
    v_ref[:] = jnp.where(sel, feat, 0.0)

    alive = jnp.ones((B, 1), _F32)
    lane_blk = jax.lax.broadcasted_iota(jnp.int32, (1, BS * B), 1)
    brow4 = jax.lax.broadcasted_iota(jnp.int32, (B, 1), 0)

    def cat4(ref, base):
        return jnp.concatenate(
            [ref[pl.ds(base + i, 1), :] for i in range(B)], axis=0)

    vprev = cat4(v_ref, 0)

    for k in range(NBLK):
        j0 = k * BS
        r0 = j0 * B
        lo = max(j0, 1)

        lstar = ls_ref[r0:r0 + BS * B, :]
        if j0 == 0:
            stat_ref[:] = jnp.concatenate(
                [lstar, jnp.full((BS * B, 1), _NEG_INF),
                 jnp.zeros((BS * B, 1), _F32)], axis=1)
            upre_ref[:] = jnp.zeros((BS * B, D), _F32)
        else:
            vpre = v_ref[0:r0, :]
            qw_blk = qw_in_ref[r0:r0 + BS * B, :]
            s_pre = _dot(qw_blk, vpre, _CONTRACT_LAST)
            lane_pre = jax.lax.broadcasted_iota(jnp.int32, (1, r0), 1)
            wpre = ((lane_pre % B == bcol[r0:r0 + BS * B, :])
                    & ((lane_pre // B).astype(_F32) >= lstar))
            m_pre = jnp.max(jnp.where(wpre, s_pre, _NEG_INF), axis=1,
                            keepdims=True)
            e_pre = jnp.where(wpre, jnp.exp(s_pre - m_pre), 0.0)
            stat_ref[:] = jnp.concatenate(
                [lstar, m_pre,
                 jnp.sum(e_pre, axis=1, keepdims=True)], axis=1)
            upre_ref[:] = _dot(e_pre, vpre, _CONTRACT_NATIVE)

        tc = lane_blk // B + j0
        tc_f = tc.astype(_F32)
        bc = lane_blk % B

        def step(j, carry):
            alive, vprev, m_ip, d_ip, e_ip, stats_cur, qw_cur = carry
            rp = (j - j0) * B
            v_blk = v_ref[r0:r0 + BS * B, :]
            u_ip = _dot(e_ip, v_blk, _CONTRACT_NATIVE)
            jn = jnp.minimum(j + 1, T - 1)
            qw_next = cat4(qw_in_ref, B * jn)
            s_next = _dot(qw_next, v_blk, _CONTRACT_LAST)

            u_pre = cat4(upre_ref, rp)
            fr = cat4(ft_in_ref, B * j)
            umv = cat4(um_ref, B * j)
            rpn = jnp.minimum(rp + B, (BS - 1) * B)
            stats_next = cat4(stat_ref, rpn)
            lstar4 = stats_cur[:, 0:1]
            m_pre = stats_cur[:, 1:2]
            d_pre = stats_cur[:, 2:3]

            corr_s = jnp.sum(qw_cur * vprev, axis=1, keepdims=True)
            corr_m = jnp.where(j > j0, corr_s, _NEG_INF)
            m_all = jnp.maximum(jnp.maximum(m_ip, corr_m), m_pre)
            c_pre = jnp.exp(m_pre - m_all)
            c_ip = jnp.exp(m_ip - m_all)
            e_c = jnp.exp(corr_m - m_all)
            num = c_pre * u_pre + c_ip * u_ip + e_c * vprev
            den = c_pre * d_pre + c_ip * d_ip + e_c
            v_att = jnp.tanh(num / den)
            alive = alive * umv
            vj = jnp.where(lstar4 >= 0.0, v_att, fr)
            vj = jnp.where(alive > 0, vj, jnp.where(kidx4 == j, fr, 0.0))
            for i in range(B):
                v_ref[pl.ds(B * j + i, 1), :] = vj[i:i + 1, :]

            lstar_n = stats_next[:, 0:1]
            wip = (bc == brow4) & (tc_f >= lstar_n) & (tc < j)
            m_ip_n = jnp.max(jnp.where(wip, s_next, _NEG_INF), axis=1,
                             keepdims=True)
            e_ip_n = jnp.where(wip, jnp.exp(s_next - m_ip_n), 0.0)
            d_ip_n = jnp.sum(e_ip_n, axis=1, keepdims=True)
            return (alive, vj, m_ip_n, d_ip_n, e_ip_n, stats_next, qw_next)

        carry = (alive, vprev,
                 jnp.full((B, 1), _NEG_INF, _F32),
                 jnp.zeros((B, 1), _F32),
                 jnp.zeros((B, BS * B), _F32),
                 cat4(stat_ref, (lo - j0) * B),
                 cat4(qw_in_ref, B * lo))
        carry = jax.lax.fori_loop(lo, j0 + BS, step, carry)
        alive, vprev = carry[0], carry[1]

    o = _dot(ft_in_ref[:], wo_ref[:], _CONTRACT_LAST) + bo_ref[:]
    out_ref[:] = jnp.maximum(o * v_ref[:], 0.0) + ft_in_ref[:]


def kernel(feature, mask_intra, umask, W_init_trans, b_init_trans,
           W_qinter, b_qinter, W_attn, b_attn, W_out, b_out):
    del b_attn
    x2 = feature.transpose(1, 0, 2).reshape(T * B, D)
    mi2 = mask_intra.astype(jnp.int32).transpose(1, 0, 2).reshape(T * B, T)
    nw = (T * B) // 16
    miT3 = mi2.reshape(nw, 16, T).transpose(0, 2, 1).reshape(nw, T * 16)
    mi0 = mask_intra.astype(jnp.int32)[:, 0, :]
    umr = umask.astype(_F32).T.reshape(T * B, 1)
    bit = b_init_trans.reshape(1, D)
    bq = b_qinter.reshape(1, D)
    bo = b_out.reshape(1, D)

    ls2 = _lstar_sc(miT3).reshape(T * B, 1)

    out2 = pl.pallas_call(
        _sse_body,
        out_shape=jax.ShapeDtypeStruct((T * B, D), _F32),
        scratch_shapes=[
            pltpu.VMEM((T * B, D), _F32),
            pltpu.VMEM((T * B, D), _F32),
            pltpu.VMEM((T * B, D), _F32),
            pltpu.VMEM((BS * B, D), _F32),
            pltpu.VMEM((BS * B, 3), _F32),
        ],
    )(x2, ls2, mi0, umr, W_init_trans, bit, W_qinter, bq, W_attn, W_out, bo)
    return out2.reshape(T, B, D).transpose(1, 0, 2)
